# Initial kernel scaffold; baseline (speedup 1.0000x reference)
import functools

import jax
import jax.numpy as jnp
from jax import lax
from jax.experimental import pallas as pl
from jax.experimental.pallas import tpu as pltpu
from jax.experimental.pallas import tpu_sc as plsc

N = 10000
E = 320000
NPAD = 10240
MID = 16
OUT = 128
GPT = 158
EPT = GPT * 128
EPAD = 16 * EPT
ROWS_PT = NPAD // 16

_mesh = plsc.VectorSubcoreMesh(core_axis_name="c", subcore_axis_name="s")


@functools.partial(
    pl.kernel,
    mesh=_mesh,
    out_type=jax.ShapeDtypeStruct((2, NPAD, MID), jnp.float32),
    scratch_types=[
        pltpu.VMEM((GPT, 128), jnp.int32),
        pltpu.VMEM((GPT, 128), jnp.int32),
        pltpu.VMEM((128, MID), jnp.float32),
        pltpu.VMEM_SHARED((NPAD, MID), jnp.float32),
        pltpu.SemaphoreType.DMA,
    ],
)
def _segsum(table, edges, out, sidx, didx, rows, acc, sem):
    c = lax.axis_index("c")
    s = lax.axis_index("s")
    zero = jnp.zeros((MID,), jnp.float32)
    for i in range(128):
        rows[i] = zero
    for j in range(ROWS_PT // 128):
        pltpu.sync_copy(rows, acc.at[pl.ds(s * ROWS_PT + j * 128, 128)])
    plsc.subcore_barrier()
    pltpu.sync_copy(edges.at[c, 0, pl.ds(s * GPT, GPT)], sidx)
    pltpu.sync_copy(edges.at[c, 1, pl.ds(s * GPT, GPT)], didx)

    def body(g, carry):
        pltpu.async_copy(table.at[sidx.at[g]], rows, sem).wait()
        pltpu.sync_copy(rows, acc.at[didx.at[g]], add=True)
        return carry

    lax.fori_loop(0, GPT, body, 0)
    plsc.subcore_barrier()
    pltpu.sync_copy(acc.at[pl.ds(s * ROWS_PT, ROWS_PT)],
                    out.at[c, pl.ds(s * ROWS_PT, ROWS_PT)])


def _tc1_body(x_ref, w_ref, cnt_ref, hs_ref, dinv_ref):
    deg = cnt_ref[0, :, :1] + 1.0
    dinv = lax.rsqrt(deg)
    h = jnp.dot(x_ref[...], w_ref[0], preferred_element_type=jnp.float32)
    hs_ref[0] = h * dinv
    dinv_ref[0] = dinv


def _tc2_body(s_ref, hs_ref, dinv_ref, b_ref, o_ref):
    dinv = dinv_ref[0]
    a = dinv * (s_ref[0] + hs_ref[0]) + b_ref[0]
    ns = jnp.where(a > 0, a, jnp.exp(a) - 1.0)
    o_ref[0] = ns * dinv


def _tc3_body(s_ref, ns_ref, dinv_ref, wb_ref, bb_ref, wg_ref, bg_ref, o_ref):
    a1 = dinv_ref[0] * (s_ref[0] + ns_ref[0])
    a2 = dinv_ref[1] * (s_ref[1] + ns_ref[1])
    f1 = jnp.dot(a1, wb_ref[0], preferred_element_type=jnp.float32) + bb_ref[0]
    f2 = jnp.dot(a2, wb_ref[1], preferred_element_type=jnp.float32) + bb_ref[1]
    z = (jnp.dot(f1, wg_ref[:OUT], preferred_element_type=jnp.float32)
         + jnp.dot(f2, wg_ref[OUT:], preferred_element_type=jnp.float32)
         + bg_ref[...])
    g = jax.nn.sigmoid(z)
    o = g * f1 + (1.0 - g) * f2
    o = o - jnp.max(o, axis=1, keepdims=True)
    o_ref[...] = o - jnp.log(jnp.sum(jnp.exp(o), axis=1, keepdims=True))


def _prescale(x, Wa, counts):
    return pl.pallas_call(
        _tc1_body,
        grid=(2,),
        in_specs=[
            pl.BlockSpec((N, 128), lambda c: (0, 0)),
            pl.BlockSpec((1, 128, MID), lambda c: (c, 0, 0)),
            pl.BlockSpec((1, N, MID), lambda c: (c, 0, 0)),
        ],
        out_specs=[
            pl.BlockSpec((1, N, MID), lambda c: (c, 0, 0)),
            pl.BlockSpec((1, N, 1), lambda c: (c, 0, 0)),
        ],
        out_shape=[
            jax.ShapeDtypeStruct((2, N, MID), jnp.float32),
            jax.ShapeDtypeStruct((2, N, 1), jnp.float32),
        ],
    )(x, Wa, counts)


def _mid_layer(S1, hs, dinv, bA):
    return pl.pallas_call(
        _tc2_body,
        grid=(2,),
        in_specs=[
            pl.BlockSpec((1, N, MID), lambda c: (c, 0, 0)),
            pl.BlockSpec((1, N, MID), lambda c: (c, 0, 0)),
            pl.BlockSpec((1, N, 1), lambda c: (c, 0, 0)),
            pl.BlockSpec((1, 1, MID), lambda c: (c, 0, 0)),
        ],
        out_specs=pl.BlockSpec((1, N, MID), lambda c: (c, 0, 0)),
        out_shape=jax.ShapeDtypeStruct((2, N, MID), jnp.float32),
    )(S1, hs, dinv, bA)


_RB = 2000


def _final(S2, ns_s, dinv, Wb, bB, Wg, bg):
    return pl.pallas_call(
        _tc3_body,
        grid=(N // _RB,),
        in_specs=[
            pl.BlockSpec((2, _RB, MID), lambda i: (0, i, 0)),
            pl.BlockSpec((2, _RB, MID), lambda i: (0, i, 0)),
            pl.BlockSpec((2, _RB, 1), lambda i: (0, i, 0)),
            pl.BlockSpec((2, MID, OUT), lambda i: (0, 0, 0)),
            pl.BlockSpec((2, 1, OUT), lambda i: (0, 0, 0)),
            pl.BlockSpec((2 * OUT, OUT), lambda i: (0, 0)),
            pl.BlockSpec((1, OUT), lambda i: (0, 0)),
        ],
        out_specs=pl.BlockSpec((_RB, OUT), lambda i: (i, 0)),
        out_shape=jax.ShapeDtypeStruct((N, OUT), jnp.float32),
    )(S2, ns_s, dinv, Wb, bB, Wg, bg)


def _prep_edges(e, off):
    src = jnp.concatenate(
        [e[0].astype(jnp.int32) + off,
         jnp.full((EPAD - E,), off + N, jnp.int32)])
    dst = jnp.concatenate(
        [e[1].astype(jnp.int32), jnp.full((EPAD - E,), N, jnp.int32)])
    return jnp.stack([src, dst]).reshape(2, 16 * GPT, 128)


def _pad_rows(a):
    return jnp.pad(a, ((0, 0), (0, NPAD - N), (0, 0))).reshape(2 * NPAD, MID)


def kernel(node_feature, adj_list, two_order_adj_list,
           W11, b11, W12, b12, W21, b21, W22, b22, Wg, bg):
    edges = jnp.stack([_prep_edges(adj_list, 0),
                       _prep_edges(two_order_adj_list, NPAD)])

    ones_col = jnp.concatenate(
        [jnp.ones((N, MID), jnp.float32), jnp.zeros((NPAD - N, MID), jnp.float32)])
    ones_tbl = jnp.concatenate([ones_col, ones_col])

    counts = _segsum(ones_tbl, edges)

    Wa = jnp.stack([W11, W21])
    hs, dinv = _prescale(node_feature, Wa, counts)

    S1 = _segsum(_pad_rows(hs), edges)

    bA = jnp.stack([b11, b21]).reshape(2, 1, MID)
    ns_s = _mid_layer(S1, hs, dinv, bA)

    S2 = _segsum(_pad_rows(ns_s), edges)

    Wb = jnp.stack([W12, W22])
    bB = jnp.stack([b12, b22]).reshape(2, 1, OUT)
    return _final(S2, ns_s, dinv, Wb, bB, Wg, bg.reshape(1, OUT))


# speedup vs baseline: 20.6875x; 20.6875x over previous
import functools

import jax
import jax.numpy as jnp
from jax import lax
from jax.experimental import pallas as pl
from jax.experimental.pallas import tpu as pltpu
from jax.experimental.pallas import tpu_sc as plsc

N = 10000
E = 320000
NPAD = 10240
MID = 16
OUT = 128
GPT = 160
EPT = GPT * 128
EPAD = 16 * EPT
ROWS_PT = NPAD // 16

@functools.cache
def _build_segsum():
    mesh = plsc.VectorSubcoreMesh(core_axis_name="c", subcore_axis_name="s")

    @functools.partial(
        pl.kernel,
        mesh=mesh,
        compiler_params=pltpu.CompilerParams(use_tc_tiling_on_sc=False),
        out_type=jax.ShapeDtypeStruct((2, NPAD, MID), jnp.float32),
        scratch_types=[
            pltpu.VMEM((GPT, 128), jnp.int32),
            pltpu.VMEM((GPT, 128), jnp.int32),
            pltpu.VMEM((128, MID), jnp.float32),
            pltpu.VMEM_SHARED((NPAD, MID), jnp.float32),
            pltpu.SemaphoreType.DMA,
        ],
    )
    def segsum(table, edges, out, sidx, didx, rows, acc, sem):
        c = lax.axis_index("c")
        s = lax.axis_index("s")
        zero = jnp.zeros((MID,), jnp.float32)
        for i in range(128):
            rows[i] = zero
        for j in range(ROWS_PT // 128):
            pltpu.sync_copy(rows, acc.at[pl.ds(s * ROWS_PT + j * 128, 128)])
        plsc.subcore_barrier()
        pltpu.sync_copy(edges.at[c, 0, pl.ds(s * GPT, GPT)], sidx)
        pltpu.sync_copy(edges.at[c, 1, pl.ds(s * GPT, GPT)], didx)

        def body(g, carry):
            pltpu.async_copy(table.at[sidx.at[g]], rows, sem).wait()
            pltpu.sync_copy(rows, acc.at[didx.at[g]], add=True)
            return carry

        lax.fori_loop(0, GPT, body, 0)
        plsc.subcore_barrier()
        pltpu.sync_copy(acc.at[pl.ds(s * ROWS_PT, ROWS_PT)],
                        out.at[c, pl.ds(s * ROWS_PT, ROWS_PT)])

    return segsum


def _tc1_body(x_ref, w_ref, cnt_ref, hs_ref, dinv_ref):
    deg = cnt_ref[0, :, :1] + 1.0
    dinv = lax.rsqrt(deg)
    h = jnp.dot(x_ref[...], w_ref[0], preferred_element_type=jnp.float32)
    hs_ref[0] = h * dinv
    dinv_ref[0] = dinv


def _tc2_body(s_ref, hs_ref, dinv_ref, b_ref, o_ref):
    dinv = dinv_ref[0]
    a = dinv * (s_ref[0] + hs_ref[0]) + b_ref[0]
    ns = jnp.where(a > 0, a, jnp.exp(a) - 1.0)
    o_ref[0] = ns * dinv


def _tc3_body(s_ref, ns_ref, dinv_ref, wb_ref, bb_ref, wg_ref, bg_ref, o_ref):
    a1 = dinv_ref[0] * (s_ref[0] + ns_ref[0])
    a2 = dinv_ref[1] * (s_ref[1] + ns_ref[1])
    f1 = jnp.dot(a1, wb_ref[0], preferred_element_type=jnp.float32) + bb_ref[0]
    f2 = jnp.dot(a2, wb_ref[1], preferred_element_type=jnp.float32) + bb_ref[1]
    z = (jnp.dot(f1, wg_ref[:OUT], preferred_element_type=jnp.float32)
         + jnp.dot(f2, wg_ref[OUT:], preferred_element_type=jnp.float32)
         + bg_ref[...])
    g = jax.nn.sigmoid(z)
    o = g * f1 + (1.0 - g) * f2
    o = o - jnp.max(o, axis=1, keepdims=True)
    o_ref[...] = o - jnp.log(jnp.sum(jnp.exp(o), axis=1, keepdims=True))


def _prescale(x, Wa, counts):
    return pl.pallas_call(
        _tc1_body,
        grid=(2,),
        in_specs=[
            pl.BlockSpec((N, 128), lambda c: (0, 0)),
            pl.BlockSpec((1, 128, MID), lambda c: (c, 0, 0)),
            pl.BlockSpec((1, N, MID), lambda c: (c, 0, 0)),
        ],
        out_specs=[
            pl.BlockSpec((1, N, MID), lambda c: (c, 0, 0)),
            pl.BlockSpec((1, N, 1), lambda c: (c, 0, 0)),
        ],
        out_shape=[
            jax.ShapeDtypeStruct((2, N, MID), jnp.float32),
            jax.ShapeDtypeStruct((2, N, 1), jnp.float32),
        ],
    )(x, Wa, counts)


def _mid_layer(S1, hs, dinv, bA):
    return pl.pallas_call(
        _tc2_body,
        grid=(2,),
        in_specs=[
            pl.BlockSpec((1, N, MID), lambda c: (c, 0, 0)),
            pl.BlockSpec((1, N, MID), lambda c: (c, 0, 0)),
            pl.BlockSpec((1, N, 1), lambda c: (c, 0, 0)),
            pl.BlockSpec((1, 1, MID), lambda c: (c, 0, 0)),
        ],
        out_specs=pl.BlockSpec((1, N, MID), lambda c: (c, 0, 0)),
        out_shape=jax.ShapeDtypeStruct((2, N, MID), jnp.float32),
    )(S1, hs, dinv, bA)


_RB = 2000


def _final(S2, ns_s, dinv, Wb, bB, Wg, bg):
    return pl.pallas_call(
        _tc3_body,
        grid=(N // _RB,),
        in_specs=[
            pl.BlockSpec((2, _RB, MID), lambda i: (0, i, 0)),
            pl.BlockSpec((2, _RB, MID), lambda i: (0, i, 0)),
            pl.BlockSpec((2, _RB, 1), lambda i: (0, i, 0)),
            pl.BlockSpec((2, MID, OUT), lambda i: (0, 0, 0)),
            pl.BlockSpec((2, 1, OUT), lambda i: (0, 0, 0)),
            pl.BlockSpec((2 * OUT, OUT), lambda i: (0, 0)),
            pl.BlockSpec((1, OUT), lambda i: (0, 0)),
        ],
        out_specs=pl.BlockSpec((_RB, OUT), lambda i: (i, 0)),
        out_shape=jax.ShapeDtypeStruct((N, OUT), jnp.float32),
    )(S2, ns_s, dinv, Wb, bB, Wg, bg)


def _prep_edges(e, off):
    src = jnp.concatenate(
        [e[0].astype(jnp.int32) + off,
         jnp.full((EPAD - E,), off + N, jnp.int32)])
    dst = jnp.concatenate(
        [e[1].astype(jnp.int32), jnp.full((EPAD - E,), N, jnp.int32)])
    return jnp.stack([src, dst]).reshape(2, 16 * GPT, 128)


def _pad_rows(a):
    return jnp.pad(a, ((0, 0), (0, NPAD - N), (0, 0))).reshape(2 * NPAD, MID)


def kernel(node_feature, adj_list, two_order_adj_list,
           W11, b11, W12, b12, W21, b21, W22, b22, Wg, bg):
    edges = jnp.stack([_prep_edges(adj_list, 0),
                       _prep_edges(two_order_adj_list, NPAD)])

    ones_col = jnp.concatenate(
        [jnp.ones((N, MID), jnp.float32), jnp.zeros((NPAD - N, MID), jnp.float32)])
    ones_tbl = jnp.concatenate([ones_col, ones_col])

    segsum = _build_segsum()
    counts = segsum(ones_tbl, edges)

    Wa = jnp.stack([W11, W21])
    hs, dinv = _prescale(node_feature, Wa, counts)

    S1 = segsum(_pad_rows(hs), edges)

    bA = jnp.stack([b11, b21]).reshape(2, 1, MID)
    ns_s = _mid_layer(S1, hs, dinv, bA)

    S2 = segsum(_pad_rows(ns_s), edges)

    Wb = jnp.stack([W12, W22])
    bB = jnp.stack([b12, b22]).reshape(2, 1, OUT)
    return _final(S2, ns_s, dinv, Wb, bB, Wg, bg.reshape(1, OUT))


# speedup vs baseline: 34.1594x; 1.6512x over previous
import functools

import jax
import jax.numpy as jnp
from jax import lax
from jax.experimental import pallas as pl
from jax.experimental.pallas import tpu as pltpu
from jax.experimental.pallas import tpu_sc as plsc

N = 10000
E = 320000
NPAD = 10240
MID = 16
OUT = 128
GPT = 160
EPT = GPT * 128
EPAD = 16 * EPT
ROWS_PT = NPAD // 16
BLK = 16
NBLK = GPT // BLK

@functools.cache
def _build_segsum():
    mesh = plsc.VectorSubcoreMesh(core_axis_name="c", subcore_axis_name="s")

    @functools.partial(
        pl.kernel,
        mesh=mesh,
        compiler_params=pltpu.CompilerParams(use_tc_tiling_on_sc=False),
        out_type=jax.ShapeDtypeStruct((2, NPAD, MID), jnp.float32),
        scratch_types=[
            pltpu.VMEM((NBLK, BLK * 128), jnp.int32),
            pltpu.VMEM((NBLK, BLK * 128), jnp.int32),
            pltpu.VMEM((2, BLK * 128, MID), jnp.float32),
            pltpu.VMEM_SHARED((NPAD, MID), jnp.float32),
            pltpu.SemaphoreType.DMA,
            pltpu.SemaphoreType.DMA,
            pltpu.SemaphoreType.DMA,
            pltpu.SemaphoreType.DMA,
        ],
    )
    def segsum(table, edges, out, sidx, didx, rows, acc, g0, g1, s0, s1):
        c = lax.axis_index("c")
        s = lax.axis_index("s")
        gsem = [g0, g1]
        ssem = [s0, s1]
        zero = jnp.zeros((MID,), jnp.float32)
        for i in range(128):
            rows[0, i] = zero
        for j in range(ROWS_PT // 128):
            pltpu.sync_copy(rows.at[0, pl.ds(0, 128)],
                            acc.at[pl.ds(s * ROWS_PT + j * 128, 128)])
        plsc.subcore_barrier()
        pltpu.sync_copy(edges.at[c, 0, s], sidx)
        pltpu.sync_copy(edges.at[c, 1, s], didx)

        gd = [None, None]
        sd = [None, None]
        gd[0] = pltpu.async_copy(table.at[sidx.at[0]], rows.at[0], gsem[0])
        for k in range(NBLK):
            b = k % 2
            gd[b].wait()
            if k + 1 < NBLK:
                if k >= 1:
                    sd[1 - b].wait()
                gd[1 - b] = pltpu.async_copy(
                    table.at[sidx.at[k + 1]], rows.at[1 - b], gsem[1 - b])
            sd[b] = pltpu.async_copy(
                rows.at[b], acc.at[didx.at[k]], ssem[b], add=True)
        sd[(NBLK - 2) % 2].wait()
        sd[(NBLK - 1) % 2].wait()
        plsc.subcore_barrier()
        pltpu.sync_copy(acc.at[pl.ds(s * ROWS_PT, ROWS_PT)],
                        out.at[c, pl.ds(s * ROWS_PT, ROWS_PT)])

    return segsum


def _tc1_body(x_ref, w_ref, cnt_ref, hs_ref, dinv_ref):
    deg = cnt_ref[0, :, :1] + 1.0
    dinv = lax.rsqrt(deg)
    h = jnp.dot(x_ref[...], w_ref[0], preferred_element_type=jnp.float32)
    hs_ref[0] = h * dinv
    dinv_ref[0] = dinv


def _tc2_body(s_ref, hs_ref, dinv_ref, b_ref, o_ref):
    dinv = dinv_ref[0]
    a = dinv * (s_ref[0] + hs_ref[0]) + b_ref[0]
    ns = jnp.where(a > 0, a, jnp.exp(a) - 1.0)
    o_ref[0] = ns * dinv


def _tc3_body(s_ref, ns_ref, dinv_ref, wb_ref, bb_ref, wg_ref, bg_ref, o_ref):
    a1 = dinv_ref[0] * (s_ref[0] + ns_ref[0])
    a2 = dinv_ref[1] * (s_ref[1] + ns_ref[1])
    f1 = jnp.dot(a1, wb_ref[0], preferred_element_type=jnp.float32) + bb_ref[0]
    f2 = jnp.dot(a2, wb_ref[1], preferred_element_type=jnp.float32) + bb_ref[1]
    z = (jnp.dot(f1, wg_ref[:OUT], preferred_element_type=jnp.float32)
         + jnp.dot(f2, wg_ref[OUT:], preferred_element_type=jnp.float32)
         + bg_ref[...])
    g = jax.nn.sigmoid(z)
    o = g * f1 + (1.0 - g) * f2
    o = o - jnp.max(o, axis=1, keepdims=True)
    o_ref[...] = o - jnp.log(jnp.sum(jnp.exp(o), axis=1, keepdims=True))


def _prescale(x, Wa, counts):
    return pl.pallas_call(
        _tc1_body,
        grid=(2,),
        in_specs=[
            pl.BlockSpec((N, 128), lambda c: (0, 0)),
            pl.BlockSpec((1, 128, MID), lambda c: (c, 0, 0)),
            pl.BlockSpec((1, N, MID), lambda c: (c, 0, 0)),
        ],
        out_specs=[
            pl.BlockSpec((1, N, MID), lambda c: (c, 0, 0)),
            pl.BlockSpec((1, N, 1), lambda c: (c, 0, 0)),
        ],
        out_shape=[
            jax.ShapeDtypeStruct((2, N, MID), jnp.float32),
            jax.ShapeDtypeStruct((2, N, 1), jnp.float32),
        ],
    )(x, Wa, counts)


def _mid_layer(S1, hs, dinv, bA):
    return pl.pallas_call(
        _tc2_body,
        grid=(2,),
        in_specs=[
            pl.BlockSpec((1, N, MID), lambda c: (c, 0, 0)),
            pl.BlockSpec((1, N, MID), lambda c: (c, 0, 0)),
            pl.BlockSpec((1, N, 1), lambda c: (c, 0, 0)),
            pl.BlockSpec((1, 1, MID), lambda c: (c, 0, 0)),
        ],
        out_specs=pl.BlockSpec((1, N, MID), lambda c: (c, 0, 0)),
        out_shape=jax.ShapeDtypeStruct((2, N, MID), jnp.float32),
    )(S1, hs, dinv, bA)


_RB = 2000


def _final(S2, ns_s, dinv, Wb, bB, Wg, bg):
    return pl.pallas_call(
        _tc3_body,
        grid=(N // _RB,),
        in_specs=[
            pl.BlockSpec((2, _RB, MID), lambda i: (0, i, 0)),
            pl.BlockSpec((2, _RB, MID), lambda i: (0, i, 0)),
            pl.BlockSpec((2, _RB, 1), lambda i: (0, i, 0)),
            pl.BlockSpec((2, MID, OUT), lambda i: (0, 0, 0)),
            pl.BlockSpec((2, 1, OUT), lambda i: (0, 0, 0)),
            pl.BlockSpec((2 * OUT, OUT), lambda i: (0, 0)),
            pl.BlockSpec((1, OUT), lambda i: (0, 0)),
        ],
        out_specs=pl.BlockSpec((_RB, OUT), lambda i: (i, 0)),
        out_shape=jax.ShapeDtypeStruct((N, OUT), jnp.float32),
    )(S2, ns_s, dinv, Wb, bB, Wg, bg)


def _prep_edges(e, off):
    src = jnp.concatenate(
        [e[0].astype(jnp.int32) + off,
         jnp.full((EPAD - E,), off + N, jnp.int32)])
    dst = jnp.concatenate(
        [e[1].astype(jnp.int32), jnp.full((EPAD - E,), N, jnp.int32)])
    return jnp.stack([src, dst]).reshape(2, 16, NBLK, BLK * 128)


def _pad_rows(a):
    return jnp.pad(a, ((0, 0), (0, NPAD - N), (0, 0))).reshape(2 * NPAD, MID)


def kernel(node_feature, adj_list, two_order_adj_list,
           W11, b11, W12, b12, W21, b21, W22, b22, Wg, bg):
    edges = jnp.stack([_prep_edges(adj_list, 0),
                       _prep_edges(two_order_adj_list, NPAD)])

    ones_col = jnp.concatenate(
        [jnp.ones((N, MID), jnp.float32), jnp.zeros((NPAD - N, MID), jnp.float32)])
    ones_tbl = jnp.concatenate([ones_col, ones_col])

    segsum = _build_segsum()
    counts = segsum(ones_tbl, edges)

    Wa = jnp.stack([W11, W21])
    hs, dinv = _prescale(node_feature, Wa, counts)

    S1 = segsum(_pad_rows(hs), edges)

    bA = jnp.stack([b11, b21]).reshape(2, 1, MID)
    ns_s = _mid_layer(S1, hs, dinv, bA)

    S2 = segsum(_pad_rows(ns_s), edges)

    Wb = jnp.stack([W12, W22])
    bB = jnp.stack([b12, b22]).reshape(2, 1, OUT)
    return _final(S2, ns_s, dinv, Wb, bB, Wg, bg.reshape(1, OUT))


# speedup vs baseline: 59.1735x; 1.7323x over previous
import functools

import jax
import jax.numpy as jnp
from jax import lax
from jax.experimental import pallas as pl
from jax.experimental.pallas import tpu as pltpu
from jax.experimental.pallas import tpu_sc as plsc

N = 10000
E = 320000
NPAD = 10240
MID = 16
OUT = 128
EPT = E // 32
CH = 2048
NCH = 5
CHS = [CH, CH, CH, CH, EPT - 4 * CH]
ROWS_PT = NPAD // 16

_SC_PARAMS = dict(
    compiler_params=pltpu.CompilerParams(use_tc_tiling_on_sc=False),
)


@functools.cache
def _build_counts():
    mesh = plsc.VectorSubcoreMesh(core_axis_name="c", subcore_axis_name="s")

    @functools.partial(
        pl.kernel,
        mesh=mesh,
        out_type=jax.ShapeDtypeStruct((2, 2, NPAD), jnp.float32),
        scratch_types=[
            pltpu.VMEM((CH,), jnp.float32),
            pltpu.VMEM((2, NCH, CH), jnp.int32),
            pltpu.VMEM_SHARED((2, NPAD), jnp.float32),
            pltpu.SemaphoreType.DMA,
        ],
        **_SC_PARAMS,
    )
    def counts(ea, eb, out, ones, didx, acc, sem):
        c = lax.axis_index("c")
        s = lax.axis_index("s")
        w = c * 16 + s
        zero = jnp.zeros((16,), jnp.float32)
        for i in range(CH // 16):
            ones[pl.ds(i * 16, 16)] = zero
        for a in range(2):
            pltpu.sync_copy(ones.at[pl.ds(0, ROWS_PT)],
                            acc.at[a, pl.ds(s * ROWS_PT, ROWS_PT)])
        one = jnp.full((16,), 1.0, jnp.float32)
        for i in range(CH // 16):
            ones[pl.ds(i * 16, 16)] = one
        for a, e in ((0, ea), (1, eb)):
            for k in range(NCH):
                pltpu.sync_copy(e.at[1, pl.ds(w * EPT + k * CH, CHS[k])],
                                didx.at[a, k, pl.ds(0, CHS[k])])
        plsc.subcore_barrier()
        for a in range(2):
            for k in range(NCH):
                pltpu.async_copy(ones.at[pl.ds(0, CHS[k])],
                                 acc.at[a].at[didx.at[a, k, pl.ds(0, CHS[k])]],
                                 sem, add=True).wait()
        plsc.subcore_barrier()
        for a in range(2):
            pltpu.sync_copy(acc.at[a, pl.ds(s * ROWS_PT, ROWS_PT)],
                            out.at[c, a, pl.ds(s * ROWS_PT, ROWS_PT)])

    return counts


@functools.cache
def _build_segsum():
    mesh = plsc.VectorSubcoreMesh(core_axis_name="c", subcore_axis_name="s")

    @functools.partial(
        pl.kernel,
        mesh=mesh,
        out_type=jax.ShapeDtypeStruct((2, 2, NPAD, MID), jnp.float32),
        scratch_types=[
            pltpu.VMEM((2, NCH, CH), jnp.int32),
            pltpu.VMEM((2, NCH, CH), jnp.int32),
            pltpu.VMEM((2, CH, MID), jnp.float32),
            pltpu.VMEM_SHARED((2, NPAD, MID), jnp.float32),
            pltpu.SemaphoreType.DMA,
            pltpu.SemaphoreType.DMA,
            pltpu.SemaphoreType.DMA,
            pltpu.SemaphoreType.DMA,
        ],
        **_SC_PARAMS,
    )
    def segsum(table, ea, eb, out, sidx, didx, rows, acc, g0, g1, s0, s1):
        c = lax.axis_index("c")
        s = lax.axis_index("s")
        w = c * 16 + s
        gsem = [g0, g1]
        ssem = [s0, s1]
        zero = jnp.zeros((MID,), jnp.float32)
        for i in range(128):
            rows[0, i] = zero
        for a in range(2):
            for j in range(ROWS_PT // 128):
                pltpu.sync_copy(
                    rows.at[0, pl.ds(0, 128)],
                    acc.at[a, pl.ds(s * ROWS_PT + j * 128, 128)])
        plsc.subcore_barrier()
        for a, e in ((0, ea), (1, eb)):
            for k in range(NCH):
                pltpu.sync_copy(e.at[0, pl.ds(w * EPT + k * CH, CHS[k])],
                                sidx.at[a, k, pl.ds(0, CHS[k])])
                pltpu.sync_copy(e.at[1, pl.ds(w * EPT + k * CH, CHS[k])],
                                didx.at[a, k, pl.ds(0, CHS[k])])

        steps = [(a, k) for a in range(2) for k in range(NCH)]
        gd = [None, None]
        sd = [None, None]

        def start_gather(j, b):
            a, k = steps[j]
            gd[b] = pltpu.async_copy(
                table.at[a].at[sidx.at[a, k, pl.ds(0, CHS[k])]],
                rows.at[b, pl.ds(0, CHS[k])], gsem[b])

        start_gather(0, 0)
        for j in range(len(steps)):
            b = j % 2
            a, k = steps[j]
            gd[b].wait()
            if j + 1 < len(steps):
                if j >= 1:
                    sd[1 - b].wait()
                start_gather(j + 1, 1 - b)
            sd[b] = pltpu.async_copy(
                rows.at[b, pl.ds(0, CHS[k])],
                acc.at[a].at[didx.at[a, k, pl.ds(0, CHS[k])]],
                ssem[b], add=True)
        sd[0].wait()
        sd[1].wait()
        plsc.subcore_barrier()
        for a in range(2):
            pltpu.sync_copy(acc.at[a, pl.ds(s * ROWS_PT, ROWS_PT)],
                            out.at[c, a, pl.ds(s * ROWS_PT, ROWS_PT)])

    return segsum


def _tc1_body(x_ref, w_ref, cnt_ref, hs_ref, dinv_ref):
    deg = cnt_ref[0, 0] + cnt_ref[1, 0] + 1.0
    dinv = lax.rsqrt(deg)
    h = jnp.dot(x_ref[...], w_ref[0], preferred_element_type=jnp.float32)
    hs_ref[0] = h * dinv
    dinv_ref[0] = dinv


def _tc2_body(s_ref, hs_ref, dinv_ref, b_ref, o_ref):
    dinv = dinv_ref[0]
    a = dinv * (s_ref[0, 0] + s_ref[1, 0] + hs_ref[0]) + b_ref[0]
    ns = jnp.where(a > 0, a, jnp.exp(a) - 1.0)
    o_ref[0] = ns * dinv


def _tc3_body(s_ref, ns_ref, dinv_ref, wb_ref, bb_ref, wg_ref, bg_ref, o_ref):
    a1 = dinv_ref[0] * (s_ref[0, 0] + s_ref[1, 0] + ns_ref[0])
    a2 = dinv_ref[1] * (s_ref[0, 1] + s_ref[1, 1] + ns_ref[1])
    f1 = jnp.dot(a1, wb_ref[0], preferred_element_type=jnp.float32) + bb_ref[0]
    f2 = jnp.dot(a2, wb_ref[1], preferred_element_type=jnp.float32) + bb_ref[1]
    z = (jnp.dot(f1, wg_ref[:OUT], preferred_element_type=jnp.float32)
         + jnp.dot(f2, wg_ref[OUT:], preferred_element_type=jnp.float32)
         + bg_ref[...])
    g = jax.nn.sigmoid(z)
    o = g * f1 + (1.0 - g) * f2
    o = o - jnp.max(o, axis=1, keepdims=True)
    o_ref[...] = o - jnp.log(jnp.sum(jnp.exp(o), axis=1, keepdims=True))


def _prescale(x, Wa, counts):
    return pl.pallas_call(
        _tc1_body,
        grid=(2,),
        in_specs=[
            pl.BlockSpec((N, 128), lambda c: (0, 0)),
            pl.BlockSpec((1, 128, MID), lambda c: (c, 0, 0)),
            pl.BlockSpec((2, 1, N, 1), lambda c: (0, c, 0, 0)),
        ],
        out_specs=[
            pl.BlockSpec((1, N, MID), lambda c: (c, 0, 0)),
            pl.BlockSpec((1, N, 1), lambda c: (c, 0, 0)),
        ],
        out_shape=[
            jax.ShapeDtypeStruct((2, N, MID), jnp.float32),
            jax.ShapeDtypeStruct((2, N, 1), jnp.float32),
        ],
    )(x, Wa, counts)


def _mid_layer(S1, hs, dinv, bA):
    return pl.pallas_call(
        _tc2_body,
        grid=(2,),
        in_specs=[
            pl.BlockSpec((2, 1, N, MID), lambda c: (0, c, 0, 0)),
            pl.BlockSpec((1, N, MID), lambda c: (c, 0, 0)),
            pl.BlockSpec((1, N, 1), lambda c: (c, 0, 0)),
            pl.BlockSpec((1, 1, MID), lambda c: (c, 0, 0)),
        ],
        out_specs=pl.BlockSpec((1, N, MID), lambda c: (c, 0, 0)),
        out_shape=jax.ShapeDtypeStruct((2, N, MID), jnp.float32),
    )(S1, hs, dinv, bA)


_RB = 2000


def _final(S2, ns_s, dinv, Wb, bB, Wg, bg):
    return pl.pallas_call(
        _tc3_body,
        grid=(N // _RB,),
        in_specs=[
            pl.BlockSpec((2, 2, _RB, MID), lambda i: (0, 0, i, 0)),
            pl.BlockSpec((2, _RB, MID), lambda i: (0, i, 0)),
            pl.BlockSpec((2, _RB, 1), lambda i: (0, i, 0)),
            pl.BlockSpec((2, MID, OUT), lambda i: (0, 0, 0)),
            pl.BlockSpec((2, 1, OUT), lambda i: (0, 0, 0)),
            pl.BlockSpec((2 * OUT, OUT), lambda i: (0, 0)),
            pl.BlockSpec((1, OUT), lambda i: (0, 0)),
        ],
        out_specs=pl.BlockSpec((_RB, OUT), lambda i: (i, 0)),
        out_shape=jax.ShapeDtypeStruct((N, OUT), jnp.float32),
    )(S2, ns_s, dinv, Wb, bB, Wg, bg)


def kernel(node_feature, adj_list, two_order_adj_list,
           W11, b11, W12, b12, W21, b21, W22, b22, Wg, bg):
    ea = adj_list.astype(jnp.int32)
    eb = two_order_adj_list.astype(jnp.int32)

    counts = _build_counts()(ea, eb)
    counts = counts[:, :, :N].reshape(2, 2, N, 1)

    segsum = _build_segsum()

    Wa = jnp.stack([W11, W21])
    hs, dinv = _prescale(node_feature, Wa, counts)

    S1 = segsum(hs, ea, eb)

    bA = jnp.stack([b11, b21]).reshape(2, 1, MID)
    ns_s = _mid_layer(S1, hs, dinv, bA)

    S2 = segsum(ns_s, ea, eb)

    Wb = jnp.stack([W12, W22])
    bB = jnp.stack([b12, b22]).reshape(2, 1, OUT)
    return _final(S2, ns_s, dinv, Wb, bB, Wg, bg.reshape(1, OUT))


# speedup vs baseline: 71.2338x; 1.2038x over previous
import functools

import jax
import jax.numpy as jnp
from jax import lax
from jax.experimental import pallas as pl
from jax.experimental.pallas import tpu as pltpu
from jax.experimental.pallas import tpu_sc as plsc

N = 10000
E = 320000
NPAD = 10240
MID = 16
OUT = 128
EPT = E // 16
CH = 2048
NCH = 10
CHS = [CH] * 9 + [EPT - 9 * CH]
ROWS_PT = NPAD // 16

_SC_PARAMS = dict(
    compiler_params=pltpu.CompilerParams(use_tc_tiling_on_sc=False),
)


@functools.cache
def _build_counts():
    mesh = plsc.VectorSubcoreMesh(core_axis_name="c", subcore_axis_name="s")
    EPW = E // 32
    KS = [CH] * 4 + [EPW - 4 * CH]

    @functools.partial(
        pl.kernel,
        mesh=mesh,
        out_type=jax.ShapeDtypeStruct((2, 2, NPAD), jnp.float32),
        scratch_types=[
            pltpu.VMEM((CH,), jnp.float32),
            pltpu.VMEM((2, 5, CH), jnp.int32),
            pltpu.VMEM_SHARED((2, NPAD), jnp.float32),
            pltpu.SemaphoreType.DMA,
        ],
        **_SC_PARAMS,
    )
    def counts(ea, eb, out, ones, didx, acc, sem):
        c = lax.axis_index("c")
        s = lax.axis_index("s")
        w = c * 16 + s
        zero = jnp.zeros((16,), jnp.float32)
        for i in range(CH // 16):
            ones[pl.ds(i * 16, 16)] = zero
        for a in range(2):
            pltpu.sync_copy(ones.at[pl.ds(0, ROWS_PT)],
                            acc.at[a, pl.ds(s * ROWS_PT, ROWS_PT)])
        one = jnp.full((16,), 1.0, jnp.float32)
        for i in range(CH // 16):
            ones[pl.ds(i * 16, 16)] = one
        for a, e in ((0, ea), (1, eb)):
            for k in range(5):
                pltpu.sync_copy(e.at[1, pl.ds(w * EPW + k * CH, KS[k])],
                                didx.at[a, k, pl.ds(0, KS[k])])
        plsc.subcore_barrier()
        for a in range(2):
            for k in range(5):
                pltpu.async_copy(ones.at[pl.ds(0, KS[k])],
                                 acc.at[a].at[didx.at[a, k, pl.ds(0, KS[k])]],
                                 sem, add=True).wait()
        plsc.subcore_barrier()
        for a in range(2):
            pltpu.sync_copy(acc.at[a, pl.ds(s * ROWS_PT, ROWS_PT)],
                            out.at[c, a, pl.ds(s * ROWS_PT, ROWS_PT)])

    return counts


@functools.cache
def _build_segsum(apply_elu):
    mesh = plsc.VectorSubcoreMesh(core_axis_name="c", subcore_axis_name="s")

    @functools.partial(
        pl.kernel,
        mesh=mesh,
        out_type=jax.ShapeDtypeStruct((2, NPAD, MID), jnp.float32),
        scratch_types=[
            pltpu.VMEM((NCH, CH), jnp.int32),
            pltpu.VMEM((NCH, CH), jnp.int32),
            pltpu.VMEM((2, CH, MID), jnp.float32),
            pltpu.VMEM((ROWS_PT,), jnp.float32),
            pltpu.VMEM((MID,), jnp.float32),
            pltpu.VMEM_SHARED((NPAD, MID), jnp.float32),
            pltpu.SemaphoreType.DMA,
            pltpu.SemaphoreType.DMA,
            pltpu.SemaphoreType.DMA,
            pltpu.SemaphoreType.DMA,
        ],
        **_SC_PARAMS,
    )
    def segsum(table, edges, dinv, bias, out,
               sidx, didx, rows, dbuf, bbuf, acc, g0, g1, s0, s1):
        c = lax.axis_index("c")
        s = lax.axis_index("s")
        gsem = [g0, g1]
        ssem = [s0, s1]
        zero = jnp.zeros((MID,), jnp.float32)
        for i in range(128):
            rows[0, i] = zero
        for j in range(ROWS_PT // 128):
            pltpu.sync_copy(rows.at[0, pl.ds(0, 128)],
                            acc.at[pl.ds(s * ROWS_PT + j * 128, 128)])
        plsc.subcore_barrier()
        for k in range(NCH):
            pltpu.sync_copy(edges.at[c, 0, pl.ds(s * EPT + k * CH, CHS[k])],
                            sidx.at[k, pl.ds(0, CHS[k])])
            pltpu.sync_copy(edges.at[c, 1, pl.ds(s * EPT + k * CH, CHS[k])],
                            didx.at[k, pl.ds(0, CHS[k])])

        gd = [None, None]
        sd = [None, None]

        def start_gather(k, b):
            gd[b] = pltpu.async_copy(
                table.at[c].at[sidx.at[k, pl.ds(0, CHS[k])]],
                rows.at[b, pl.ds(0, CHS[k])], gsem[b])

        start_gather(0, 0)
        for k in range(NCH):
            b = k % 2
            gd[b].wait()
            if k + 1 < NCH:
                if k >= 1:
                    sd[1 - b].wait()
                start_gather(k + 1, 1 - b)
            sd[b] = pltpu.async_copy(
                rows.at[b, pl.ds(0, CHS[k])],
                acc.at[didx.at[k, pl.ds(0, CHS[k])]],
                ssem[b], add=True)
        sd[0].wait()
        sd[1].wait()
        plsc.subcore_barrier()

        r0 = s * ROWS_PT
        pltpu.sync_copy(acc.at[pl.ds(r0, ROWS_PT)],
                        rows.at[0, pl.ds(0, ROWS_PT)])
        pltpu.sync_copy(table.at[c, pl.ds(r0, ROWS_PT)],
                        rows.at[1, pl.ds(0, ROWS_PT)])
        pltpu.sync_copy(dinv.at[c, pl.ds(r0, ROWS_PT)], dbuf)
        if apply_elu:
            pltpu.sync_copy(bias.at[c], bbuf)
            brow = bbuf[...]

        def body(g, carry):
            base = g * 16
            dvec = dbuf[pl.ds(base, 16)]
            for j in range(16):
                d = dvec[j]
                a = d * (rows[0, base + j] + rows[1, base + j])
                if apply_elu:
                    a = a + brow
                    a = jnp.where(a > 0, a, jnp.exp(a) - 1.0) * d
                rows[0, base + j] = a
            return carry

        lax.fori_loop(0, ROWS_PT // 16, body, 0)
        pltpu.sync_copy(rows.at[0, pl.ds(0, ROWS_PT)],
                        out.at[c, pl.ds(r0, ROWS_PT)])

    return segsum


def _tc1_body(x_ref, w_ref, cnt_ref, hs_ref, dinv_ref):
    deg = cnt_ref[0, 0] + cnt_ref[1, 0] + 1.0
    dinv = lax.rsqrt(deg)
    h = jnp.dot(x_ref[...], w_ref[0], preferred_element_type=jnp.float32)
    hs_ref[0] = h * dinv
    dinv_ref[0] = dinv


def _tc3_body(a_ref, wb_ref, bb_ref, wg_ref, bg_ref, o_ref):
    f1 = jnp.dot(a_ref[0], wb_ref[0], preferred_element_type=jnp.float32) + bb_ref[0]
    f2 = jnp.dot(a_ref[1], wb_ref[1], preferred_element_type=jnp.float32) + bb_ref[1]
    z = (jnp.dot(f1, wg_ref[:OUT], preferred_element_type=jnp.float32)
         + jnp.dot(f2, wg_ref[OUT:], preferred_element_type=jnp.float32)
         + bg_ref[...])
    g = jax.nn.sigmoid(z)
    o = g * f1 + (1.0 - g) * f2
    o = o - jnp.max(o, axis=1, keepdims=True)
    o_ref[...] = o - jnp.log(jnp.sum(jnp.exp(o), axis=1, keepdims=True))


def _prescale(x, Wa, counts):
    return pl.pallas_call(
        _tc1_body,
        grid=(2,),
        in_specs=[
            pl.BlockSpec((N, 128), lambda c: (0, 0)),
            pl.BlockSpec((1, 128, MID), lambda c: (c, 0, 0)),
            pl.BlockSpec((2, 1, N, 1), lambda c: (0, c, 0, 0)),
        ],
        out_specs=[
            pl.BlockSpec((1, N, MID), lambda c: (c, 0, 0)),
            pl.BlockSpec((1, N, 1), lambda c: (c, 0, 0)),
        ],
        out_shape=[
            jax.ShapeDtypeStruct((2, NPAD, MID), jnp.float32),
            jax.ShapeDtypeStruct((2, NPAD, 1), jnp.float32),
        ],
    )(x, Wa, counts)


_RB = 2000


def _final(a12, Wb, bB, Wg, bg):
    return pl.pallas_call(
        _tc3_body,
        grid=(N // _RB,),
        in_specs=[
            pl.BlockSpec((2, _RB, MID), lambda i: (0, i, 0)),
            pl.BlockSpec((2, MID, OUT), lambda i: (0, 0, 0)),
            pl.BlockSpec((2, 1, OUT), lambda i: (0, 0, 0)),
            pl.BlockSpec((2 * OUT, OUT), lambda i: (0, 0)),
            pl.BlockSpec((1, OUT), lambda i: (0, 0)),
        ],
        out_specs=pl.BlockSpec((_RB, OUT), lambda i: (i, 0)),
        out_shape=jax.ShapeDtypeStruct((N, OUT), jnp.float32),
    )(a12, Wb, bB, Wg, bg)


def kernel(node_feature, adj_list, two_order_adj_list,
           W11, b11, W12, b12, W21, b21, W22, b22, Wg, bg):
    ea = adj_list.astype(jnp.int32)
    eb = two_order_adj_list.astype(jnp.int32)
    edges = jnp.stack([ea, eb])

    counts = _build_counts()(ea, eb)
    counts = counts[:, :, :N].reshape(2, 2, N, 1)

    Wa = jnp.stack([W11, W21])
    hs, dinv = _prescale(node_feature, Wa, counts)
    dinv2 = dinv.reshape(2, NPAD)
    bA = jnp.stack([b11, b21])

    ns_s = _build_segsum(True)(hs, edges, dinv2, bA)
    a12 = _build_segsum(False)(ns_s, edges, dinv2, bA)

    Wb = jnp.stack([W12, W22])
    bB = jnp.stack([b12, b22]).reshape(2, 1, OUT)
    return _final(a12, Wb, bB, Wg, bg.reshape(1, OUT))


# speedup vs baseline: 88.2135x; 1.2384x over previous
import functools

import jax
import jax.numpy as jnp
from jax import lax
from jax.experimental import pallas as pl
from jax.experimental.pallas import tpu as pltpu
from jax.experimental.pallas import tpu_sc as plsc

N = 10000
E = 320000
NPAD = 10240
MID = 16
OUT = 128
EPT = E // 16
CH = 2048
NCH = 10
CHS = [CH] * 9 + [EPT - 9 * CH]
NBUF = 2
LAG = 1
ROWS_PT = NPAD // 16

_SC_PARAMS = dict(
    compiler_params=pltpu.CompilerParams(use_tc_tiling_on_sc=False),
)


def _newton_rsqrt(x):
    i = lax.bitcast_convert_type(x, jnp.int32)
    i = 0x5F3759DF - lax.shift_right_arithmetic(i, 1)
    y = lax.bitcast_convert_type(i, jnp.float32)
    for _ in range(3):
        y = y * (1.5 - 0.5 * x * y * y)
    return y


@functools.cache
def _build_counts():
    mesh = plsc.VectorSubcoreMesh(core_axis_name="c", subcore_axis_name="s")

    @functools.partial(
        pl.kernel,
        mesh=mesh,
        out_type=jax.ShapeDtypeStruct((2, NPAD), jnp.float32),
        scratch_types=[
            pltpu.VMEM((CH,), jnp.float32),
            pltpu.VMEM((NCH, CH), jnp.int32),
            pltpu.VMEM((ROWS_PT,), jnp.float32),
            pltpu.VMEM_SHARED((NPAD,), jnp.float32),
            pltpu.SemaphoreType.DMA,
            pltpu.SemaphoreType.DMA,
        ],
        **_SC_PARAMS,
    )
    def counts(edges, out, ones, didx, dbuf, acc, sem, sem2):
        c = lax.axis_index("c")
        s = lax.axis_index("s")
        zero = jnp.zeros((16,), jnp.float32)
        for i in range(CH // 16):
            ones[pl.ds(i * 16, 16)] = zero
        pltpu.sync_copy(ones.at[pl.ds(0, ROWS_PT)],
                        acc.at[pl.ds(s * ROWS_PT, ROWS_PT)])
        one = jnp.full((16,), 1.0, jnp.float32)
        for i in range(CH // 16):
            ones[pl.ds(i * 16, 16)] = one
        for k in range(NCH):
            pltpu.sync_copy(edges.at[c, 1, pl.ds(s * EPT + k * CH, CHS[k])],
                            didx.at[k, pl.ds(0, CHS[k])])
        plsc.subcore_barrier()
        sems = [sem, sem2]
        prev = None
        for k in range(NCH):
            cur = pltpu.async_copy(
                ones.at[pl.ds(0, CHS[k])],
                acc.at[didx.at[k, pl.ds(0, CHS[k])]],
                sems[k % 2], add=True)
            if prev is not None:
                prev.wait()
            prev = cur
        prev.wait()
        plsc.subcore_barrier()
        pltpu.sync_copy(acc.at[pl.ds(s * ROWS_PT, ROWS_PT)], dbuf)

        def body(g, carry):
            base = g * 16
            deg = dbuf[pl.ds(base, 16)] + 1.0
            dbuf[pl.ds(base, 16)] = _newton_rsqrt(deg)
            return carry

        lax.fori_loop(0, ROWS_PT // 16, body, 0)
        pltpu.sync_copy(dbuf, out.at[c, pl.ds(s * ROWS_PT, ROWS_PT)])

    return counts


@functools.cache
def _build_segsum(apply_elu):
    mesh = plsc.VectorSubcoreMesh(core_axis_name="c", subcore_axis_name="s")

    @functools.partial(
        pl.kernel,
        mesh=mesh,
        out_type=jax.ShapeDtypeStruct((2, NPAD, MID), jnp.float32),
        scratch_types=[
            pltpu.VMEM((NCH, CH), jnp.int32),
            pltpu.VMEM((NCH, CH), jnp.int32),
            pltpu.VMEM((NBUF, CH, MID), jnp.float32),
            pltpu.VMEM((ROWS_PT,), jnp.float32),
            pltpu.VMEM((MID,), jnp.float32),
            pltpu.VMEM_SHARED((NPAD, MID), jnp.float32),
            pltpu.VMEM_SHARED((NPAD, MID), jnp.float32),
            pltpu.SemaphoreType.DMA,
            pltpu.SemaphoreType.DMA,
            pltpu.SemaphoreType.DMA,
            pltpu.SemaphoreType.DMA,
        ],
        **_SC_PARAMS,
    )
    def segsum(table, edges, dinv, bias, out,
               sidx, didx, rows, dbuf, bbuf, acc, tbl,
               g0, g1, s0, s1):
        c = lax.axis_index("c")
        s = lax.axis_index("s")
        gsem = [g0, g1]
        ssem = [s0, s1]
        r0 = s * ROWS_PT
        pltpu.sync_copy(dinv.at[c, pl.ds(r0, ROWS_PT)], dbuf)
        if apply_elu:
            pltpu.sync_copy(table.at[c, pl.ds(r0, ROWS_PT)],
                            rows.at[1, pl.ds(0, ROWS_PT)])

            def scale_body(g, carry):
                base = g * 16
                dvec = dbuf[pl.ds(base, 16)]
                for j in range(16):
                    rows[1, base + j] = rows[1, base + j] * dvec[j]
                return carry

            lax.fori_loop(0, ROWS_PT // 16, scale_body, 0)
            pltpu.sync_copy(rows.at[1, pl.ds(0, ROWS_PT)],
                            tbl.at[pl.ds(r0, ROWS_PT)])
        else:
            pltpu.sync_copy(table.at[c, pl.ds(r0, ROWS_PT)],
                            tbl.at[pl.ds(r0, ROWS_PT)])
        zero = jnp.zeros((MID,), jnp.float32)
        for i in range(128):
            rows[0, i] = zero
        for j in range(ROWS_PT // 128):
            pltpu.sync_copy(rows.at[0, pl.ds(0, 128)],
                            acc.at[pl.ds(s * ROWS_PT + j * 128, 128)])
        plsc.subcore_barrier()
        for k in range(NCH):
            pltpu.sync_copy(edges.at[c, 0, pl.ds(s * EPT + k * CH, CHS[k])],
                            sidx.at[k, pl.ds(0, CHS[k])])
            pltpu.sync_copy(edges.at[c, 1, pl.ds(s * EPT + k * CH, CHS[k])],
                            didx.at[k, pl.ds(0, CHS[k])])

        gd = [None, None]
        sd = [None, None]

        def start_gather(k, b):
            gd[b] = pltpu.async_copy(
                tbl.at[sidx.at[k, pl.ds(0, CHS[k])]],
                rows.at[b, pl.ds(0, CHS[k])], gsem[b])

        start_gather(0, 0)
        for k in range(NCH):
            b = k % 2
            gd[b].wait()
            if k + 1 < NCH:
                if k >= 1:
                    sd[1 - b].wait()
                start_gather(k + 1, 1 - b)
            sd[b] = pltpu.async_copy(
                rows.at[b, pl.ds(0, CHS[k])],
                acc.at[didx.at[k, pl.ds(0, CHS[k])]],
                ssem[b], add=True)
        sd[0].wait()
        sd[1].wait()
        plsc.subcore_barrier()

        pltpu.sync_copy(acc.at[pl.ds(r0, ROWS_PT)],
                        rows.at[0, pl.ds(0, ROWS_PT)])
        pltpu.sync_copy(tbl.at[pl.ds(r0, ROWS_PT)],
                        rows.at[1, pl.ds(0, ROWS_PT)])
        if apply_elu:
            pltpu.sync_copy(bias.at[c], bbuf)
            brow = bbuf[...]

        def body(g, carry):
            base = g * 16
            dvec = dbuf[pl.ds(base, 16)]
            for j in range(16):
                d = dvec[j]
                a = d * (rows[0, base + j] + rows[1, base + j])
                if apply_elu:
                    a = a + brow
                    a = jnp.where(a > 0, a, jnp.exp(a) - 1.0) * d
                rows[0, base + j] = a
            return carry

        lax.fori_loop(0, ROWS_PT // 16, body, 0)
        pltpu.sync_copy(rows.at[0, pl.ds(0, ROWS_PT)],
                        out.at[c, pl.ds(r0, ROWS_PT)])

    return segsum


def _tcmm_body(x_ref, w_ref, h_ref):
    h_ref[0] = jnp.dot(x_ref[...], w_ref[0],
                       preferred_element_type=jnp.float32)


def _tc3_body(a_ref, wb_ref, bb_ref, wg_ref, bg_ref, o_ref):
    f1 = jnp.dot(a_ref[0], wb_ref[0], preferred_element_type=jnp.float32) + bb_ref[0]
    f2 = jnp.dot(a_ref[1], wb_ref[1], preferred_element_type=jnp.float32) + bb_ref[1]
    z = (jnp.dot(f1, wg_ref[:OUT], preferred_element_type=jnp.float32)
         + jnp.dot(f2, wg_ref[OUT:], preferred_element_type=jnp.float32)
         + bg_ref[...])
    g = jax.nn.sigmoid(z)
    o = g * f1 + (1.0 - g) * f2
    o = o - jnp.max(o, axis=1, keepdims=True)
    o_ref[...] = o - jnp.log(jnp.sum(jnp.exp(o), axis=1, keepdims=True))


def _matmul_in(x, Wa):
    return pl.pallas_call(
        _tcmm_body,
        grid=(2,),
        in_specs=[
            pl.BlockSpec((N, 128), lambda c: (0, 0)),
            pl.BlockSpec((1, 128, MID), lambda c: (c, 0, 0)),
        ],
        out_specs=pl.BlockSpec((1, N, MID), lambda c: (c, 0, 0)),
        out_shape=jax.ShapeDtypeStruct((2, NPAD, MID), jnp.float32),
    )(x, Wa)


_RB = 2000


def _final(a12, Wb, bB, Wg, bg):
    return pl.pallas_call(
        _tc3_body,
        grid=(N // _RB,),
        in_specs=[
            pl.BlockSpec((2, _RB, MID), lambda i: (0, i, 0)),
            pl.BlockSpec((2, MID, OUT), lambda i: (0, 0, 0)),
            pl.BlockSpec((2, 1, OUT), lambda i: (0, 0, 0)),
            pl.BlockSpec((2 * OUT, OUT), lambda i: (0, 0)),
            pl.BlockSpec((1, OUT), lambda i: (0, 0)),
        ],
        out_specs=pl.BlockSpec((_RB, OUT), lambda i: (i, 0)),
        out_shape=jax.ShapeDtypeStruct((N, OUT), jnp.float32),
    )(a12, Wb, bB, Wg, bg)


def kernel(node_feature, adj_list, two_order_adj_list,
           W11, b11, W12, b12, W21, b21, W22, b22, Wg, bg):
    ea = adj_list.astype(jnp.int32)
    eb = two_order_adj_list.astype(jnp.int32)
    edges = jnp.stack([ea, eb])

    dinv = _build_counts()(edges)
    Wa = jnp.stack([W11, W21])
    h = _matmul_in(node_feature, Wa)
    bA = jnp.stack([b11, b21])

    ns_s = _build_segsum(True)(h, edges, dinv, bA)
    a12 = _build_segsum(False)(ns_s, edges, dinv, bA)

    Wb = jnp.stack([W12, W22])
    bB = jnp.stack([b12, b22]).reshape(2, 1, OUT)
    return _final(a12, Wb, bB, Wg, bg.reshape(1, OUT))
